# baseline (speedup 1.0000x reference)
import jax
import jax.numpy as jnp
from jax.experimental import pallas as pl

_PROB = 0.12


def _jitter_shift(T):
    k1, k2 = jax.random.split(jax.random.key(42))
    replace = jax.random.bernoulli(k1, _PROB, (T,))
    direction = jnp.where(jax.random.bernoulli(k2, 0.5, (T,)), 1, -1)
    idx = jnp.arange(T)
    direction = jnp.where(idx == 0, 1, direction)
    direction = jnp.where(idx == T - 1, -1, direction)
    return jnp.where(replace, direction, 0).astype(jnp.int32)


def _jitter_body(d_ref, x_ref, o_ref):
    x = x_ref[...]
    d = d_ref[...]
    xl = jnp.roll(x, -1, axis=1)
    xr = jnp.roll(x, 1, axis=1)
    o_ref[...] = jnp.where(d == 1, xl, jnp.where(d == -1, xr, x))


def kernel(quantized):
    B, C, T = quantized.shape
    d = _jitter_shift(T).reshape(1, T)
    x = quantized.reshape(B * C, T)
    rows = B * C
    block_rows = 256
    grid = (rows // block_rows,)
    out = pl.pallas_call(
        _jitter_body,
        grid=grid,
        in_specs=[
            pl.BlockSpec((1, T), lambda i: (0, 0)),
            pl.BlockSpec((block_rows, T), lambda i: (i, 0)),
        ],
        out_specs=pl.BlockSpec((block_rows, T), lambda i: (i, 0)),
        out_shape=jax.ShapeDtypeStruct((rows, T), quantized.dtype),
    )(d, x)
    return out.reshape(B, C, T)


# speedup vs baseline: 1.0758x; 1.0758x over previous
import jax
import jax.numpy as jnp
from jax.experimental import pallas as pl

_PROB = 0.12


def _jitter_shift(T):
    k1, k2 = jax.random.split(jax.random.key(42))
    replace = jax.random.bernoulli(k1, _PROB, (T,))
    direction = jnp.where(jax.random.bernoulli(k2, 0.5, (T,)), 1, -1)
    idx = jnp.arange(T)
    direction = jnp.where(idx == 0, 1, direction)
    direction = jnp.where(idx == T - 1, -1, direction)
    return jnp.where(replace, direction, 0).astype(jnp.int32)


def _jitter_body(d_ref, x_ref, o_ref):
    x = x_ref[...]
    d = d_ref[...]
    xl = jnp.roll(x, -1, axis=1)
    xr = jnp.roll(x, 1, axis=1)
    o_ref[...] = jnp.where(d == 1, xl, jnp.where(d == -1, xr, x))


def kernel(quantized):
    B, C, T = quantized.shape
    d = _jitter_shift(T).reshape(1, T)
    x = quantized.reshape(B * C, T)
    rows = B * C
    block_rows = 512
    grid = (rows // block_rows,)
    out = pl.pallas_call(
        _jitter_body,
        grid=grid,
        in_specs=[
            pl.BlockSpec((1, T), lambda i: (0, 0)),
            pl.BlockSpec((block_rows, T), lambda i: (i, 0)),
        ],
        out_specs=pl.BlockSpec((block_rows, T), lambda i: (i, 0)),
        out_shape=jax.ShapeDtypeStruct((rows, T), quantized.dtype),
    )(d, x)
    return out.reshape(B, C, T)
